# baseline (speedup 1.0000x reference)
import functools

import jax
import jax.numpy as jnp
from jax import lax
from jax.experimental import pallas as pl
from jax.experimental.pallas import tpu as pltpu

N = 10000
E = 320000
G = 64
NPAD = 10240


def _head_body(h_ref, batch_ref, wd1_ref, bd1_ref, gamma_ref, beta_ref,
               wd2_ref, bd2_ref, out_ref):
    h = h_ref[...]
    b = batch_ref[...]
    gids = lax.broadcasted_iota(jnp.int32, (G, NPAD), 0)
    onehot = (b == gids).astype(jnp.float32)
    sums = jnp.dot(onehot, h, preferred_element_type=jnp.float32, precision=lax.Precision.HIGHEST)
    cnts = jnp.sum(onehot, axis=1, keepdims=True)
    p = sums / jnp.maximum(cnts, 1.0)
    p = jnp.dot(p, wd1_ref[...], preferred_element_type=jnp.float32) + bd1_ref[...]
    mean = jnp.mean(p, axis=0, keepdims=True)
    var = jnp.mean((p - mean) ** 2, axis=0, keepdims=True)
    p = (p - mean) / jnp.sqrt(var + 1e-5) * gamma_ref[...] + beta_ref[...]
    p = jnp.where(p >= 0, p, 0.1 * p)
    out_ref[...] = jnp.dot(p, wd2_ref[...], preferred_element_type=jnp.float32) + bd2_ref[...]


def _head(h, batch, Wd1, bd1, gamma, beta, Wd2, bd2):
    h_pad = jnp.zeros((NPAD, h.shape[1]), h.dtype).at[:N].set(h)
    b_pad = jnp.full((1, NPAD), G, jnp.int32).at[0, :N].set(batch)
    out = pl.pallas_call(
        _head_body,
        out_shape=jax.ShapeDtypeStruct((G, Wd2.shape[1]), jnp.float32),
    )(h_pad, b_pad, Wd1, bd1.reshape(1, -1), gamma.reshape(1, -1),
      beta.reshape(1, -1), Wd2, bd2.reshape(1, -1))
    return out


def _gatv2(x, edge_index, edge_attr, Wl, bl, Wr, br, We, att, bias):
    src = edge_index[0]
    dst = edge_index[1]
    n = x.shape[0]
    xl = x @ Wl + bl
    xr = x @ Wr + br
    xj = jnp.take(xl, src, axis=0)
    xi = jnp.take(xr, dst, axis=0)
    m = xj + xi + edge_attr @ We
    m = jax.nn.leaky_relu(m, negative_slope=0.2)
    logits = jnp.sum(m * att, axis=-1)
    lmax = jax.ops.segment_max(logits, dst, num_segments=n)
    lmax = jnp.where(jnp.isfinite(lmax), lmax, 0.0)
    ex = jnp.exp(logits - jnp.take(lmax, dst))
    den = jax.ops.segment_sum(ex, dst, num_segments=n)
    alpha = ex / (jnp.take(den, dst) + 1e-16)
    out = jax.ops.segment_sum(xj * alpha[:, None], dst, num_segments=n)
    return out + bias


def kernel(x, edge_index, edge_attr, batch,
           Wl1, bl1, Wr1, br1, We1, att1, bias1,
           Wl2, bl2, Wr2, br2, We2, att2, bias2,
           Wd1, bd1, gamma, beta, Wd2, bd2):
    h = jax.nn.relu(_gatv2(x, edge_index, edge_attr, Wl1, bl1, Wr1, br1, We1, att1, bias1))
    h = jax.nn.relu(_gatv2(h, edge_index, edge_attr, Wl2, bl2, Wr2, br2, We2, att2, bias2))
    return _head(h, batch, Wd1, bd1, gamma, beta, Wd2, bd2)


# speedup vs baseline: 3.4696x; 3.4696x over previous
import functools

import jax
import jax.numpy as jnp
from jax import lax
from jax.experimental import pallas as pl
from jax.experimental.pallas import tpu as pltpu
from jax.experimental.pallas import tpu_sc as plsc

N = 10000
E = 320000
G = 64
NPAD = 10240
NW = 32
B = 80
EPW = E // NW
EPT = E // 16

_mesh = lambda: plsc.VectorSubcoreMesh(core_axis_name="c", subcore_axis_name="s")


def _dyn_gather(v, idx):
    return lax.gather(
        v, idx[:, None],
        dimension_numbers=lax.GatherDimensionNumbers(
            offset_dims=(), collapsed_slice_dims=(0,), start_index_map=(0,)),
        slice_sizes=(1,),
        mode=lax.GatherScatterMode.PROMISE_IN_BOUNDS)


def _lane_sum(v, lane):
    for off in (8, 4, 2, 1):
        v = v + _dyn_gather(v, lax.bitwise_xor(lane, off))
    return v


def _node_mm1_body(x_ref, wl_ref, wr_ref, bl_ref, br_ref,
                   xlf, xrf, xllo, xlhi):
    xb = x_ref[...]
    xl = jnp.dot(xb, wl_ref[...], preferred_element_type=jnp.float32) + bl_ref[...]
    xr = jnp.dot(xb, wr_ref[...], preferred_element_type=jnp.float32) + br_ref[...]
    h = xl.shape[1] // 2
    xlf[...] = xl
    xrf[...] = xr
    xllo[...] = xl[:, :h]
    xlhi[...] = xl[:, h:]


def _node_mm1(x_pad, Wl, bl, Wr, br):
    D, H = Wl.shape
    Hh = H // 2
    R = 1024
    return pl.pallas_call(
        _node_mm1_body,
        grid=(NPAD // R,),
        in_specs=[
            pl.BlockSpec((R, D), lambda i: (i, 0)),
            pl.BlockSpec((D, H), lambda i: (0, 0)),
            pl.BlockSpec((D, H), lambda i: (0, 0)),
            pl.BlockSpec((1, H), lambda i: (0, 0)),
            pl.BlockSpec((1, H), lambda i: (0, 0)),
        ],
        out_specs=[pl.BlockSpec((R, H), lambda i: (i, 0))] * 2
        + [pl.BlockSpec((R, Hh), lambda i: (i, 0))] * 2,
        out_shape=[jax.ShapeDtypeStruct((NPAD, H), jnp.float32)] * 2
        + [jax.ShapeDtypeStruct((NPAD, Hh), jnp.float32)] * 2,
    )(x_pad, Wl, Wr, bl.reshape(1, -1), br.reshape(1, -1))


def _node_mm2_body(ha_ref, hb_ref, wla_ref, wlb_ref, wra_ref, wrb_ref,
                   bl_ref, br_ref, xlf, xrf):
    ha = ha_ref[...]
    hb = hb_ref[...]
    xlf[...] = (jnp.dot(ha, wla_ref[...], preferred_element_type=jnp.float32)
                + jnp.dot(hb, wlb_ref[...], preferred_element_type=jnp.float32)
                + bl_ref[...])
    xrf[...] = (jnp.dot(ha, wra_ref[...], preferred_element_type=jnp.float32)
                + jnp.dot(hb, wrb_ref[...], preferred_element_type=jnp.float32)
                + br_ref[...])


def _node_mm2(h_lo, h_hi, Wl, bl, Wr, br):
    Dh = h_lo.shape[1]
    H = Wl.shape[1]
    R = 1024
    return pl.pallas_call(
        _node_mm2_body,
        grid=(NPAD // R,),
        in_specs=[
            pl.BlockSpec((R, Dh), lambda i: (i, 0)),
            pl.BlockSpec((R, Dh), lambda i: (i, 0)),
            pl.BlockSpec((Dh, H), lambda i: (0, 0)),
            pl.BlockSpec((Dh, H), lambda i: (0, 0)),
            pl.BlockSpec((Dh, H), lambda i: (0, 0)),
            pl.BlockSpec((Dh, H), lambda i: (0, 0)),
            pl.BlockSpec((1, H), lambda i: (0, 0)),
            pl.BlockSpec((1, H), lambda i: (0, 0)),
        ],
        out_specs=[pl.BlockSpec((R, H), lambda i: (i, 0))] * 2,
        out_shape=[jax.ShapeDtypeStruct((NPAD, H), jnp.float32)] * 2,
    )(h_lo, h_hi, Wl[:Dh], Wl[Dh:], Wr[:Dh], Wr[Dh:],
      bl.reshape(1, -1), br.reshape(1, -1))


def _edge_mm_body(ea_ref, we_ref, out_ref):
    out_ref[...] = jnp.dot(ea_ref[...], we_ref[...],
                           preferred_element_type=jnp.float32)


def _edge_mm(edge_attr, We):
    De, H = We.shape
    R = 3200
    return pl.pallas_call(
        _edge_mm_body,
        grid=(E // R,),
        in_specs=[
            pl.BlockSpec((R, De), lambda i: (i, 0)),
            pl.BlockSpec((De, H), lambda i: (0, 0)),
        ],
        out_specs=pl.BlockSpec((R, H), lambda i: (i, 0)),
        out_shape=jax.ShapeDtypeStruct((E, H), jnp.float32),
    )(edge_attr, We)


def _sc_pass1_body(H, src_h, dst_h, xl_h, xr_h, ew_h,
                   att_h, ex_h, den_h,
                   src_v, dst_v, xj_v, xi_v, ew_v, att_v, ex_v,
                   zb, den_sh, sem):
    nk = H // 16
    c = lax.axis_index("c")
    s = lax.axis_index("s")
    wid = c * 16 + s
    for z in range(40):
        zb[pl.ds(z * 16, 16)] = jnp.zeros((16,), jnp.float32)
    pltpu.sync_copy(zb, den_sh.at[pl.ds(s * 640, 640)])
    pltpu.sync_copy(att_h, att_v)
    plsc.subcore_barrier()
    attk = [att_v[pl.ds(k * 16, 16)] for k in range(nk)]
    lane = lax.iota(jnp.int32, 16)
    base = wid * EPW

    def chunk(i, carry):
        eb = base + i * B
        pltpu.sync_copy(src_h.at[pl.ds(eb, B)], src_v)
        pltpu.sync_copy(dst_h.at[pl.ds(eb, B)], dst_v)
        cps = [
            pltpu.async_copy(xl_h.at[src_v], xj_v, sem),
            pltpu.async_copy(xr_h.at[dst_v], xi_v, sem),
            pltpu.async_copy(ew_h.at[pl.ds(eb, B), :], ew_v, sem),
        ]
        for cp in cps:
            cp.wait()

        def group(g, gcarry):
            lvec = jnp.zeros((16,), jnp.float32)
            for j in range(16):
                e = g * 16 + j
                acc = jnp.zeros((16,), jnp.float32)
                for k in range(nk):
                    v = (xj_v[e, pl.ds(k * 16, 16)]
                         + xi_v[e, pl.ds(k * 16, 16)]
                         + ew_v[e, pl.ds(k * 16, 16)])
                    m = jnp.maximum(v, 0.2 * v)
                    acc = acc + m * attk[k]
                lvec = jnp.where(lane == j, _lane_sum(acc, lane), lvec)
            ex_v[pl.ds(g * 16, 16)] = jnp.exp(lvec)
            return gcarry

        lax.fori_loop(0, B // 16, group, 0)
        pltpu.sync_copy(ex_v, ex_h.at[pl.ds(eb, B)])
        pltpu.sync_copy(ex_v, den_sh.at[dst_v], add=True)
        return carry

    lax.fori_loop(0, EPW // B, chunk, 0)
    plsc.subcore_barrier()
    pltpu.sync_copy(den_sh.at[pl.ds(s * 640, 640)], zb)
    pltpu.sync_copy(zb, den_h.at[c, pl.ds(s * 640, 640)])


def _sc_pass1(src, dst, xl, xr, eW, att):
    H = eW.shape[1]
    kern = functools.partial(
        pl.kernel,
        mesh=_mesh(),
        compiler_params=pltpu.CompilerParams(needs_layout_passes=False),
        out_type=[
            jax.ShapeDtypeStruct((E,), jnp.float32),
            jax.ShapeDtypeStruct((2, NPAD), jnp.float32),
        ],
        scratch_types=[
            pltpu.VMEM((B,), jnp.int32),
            pltpu.VMEM((B,), jnp.int32),
            pltpu.VMEM((B, H), jnp.float32),
            pltpu.VMEM((B, H), jnp.float32),
            pltpu.VMEM((B, H), jnp.float32),
            pltpu.VMEM((H,), jnp.float32),
            pltpu.VMEM((B,), jnp.float32),
            pltpu.VMEM((640,), jnp.float32),
            pltpu.VMEM_SHARED((NPAD,), jnp.float32),
            pltpu.SemaphoreType.DMA,
        ],
    )
    return kern(functools.partial(_sc_pass1_body, H))(src, dst, xl, xr, eW, att)


def _sc_pass2_body(Hh, src_h, dst_h, ex_h, den2_h, xllo_h, xlhi_h, bias_h,
                   hlo_h, hhi_h,
                   src_v, dst_v, ex_c, xj, d0, d1, wb, bias_v, acc_sh, sem):
    nkh = Hh // 16
    c = lax.axis_index("c")
    s = lax.axis_index("s")
    lane = lax.iota(jnp.int32, 16)

    def zrow(r, carry):
        for k in range(nkh):
            wb[r, pl.ds(k * 16, 16)] = jnp.zeros((16,), jnp.float32)
        return carry

    lax.fori_loop(0, 64, zrow, 0)
    for z in range(10):
        pltpu.sync_copy(wb, acc_sh.at[pl.ds(s * 640 + z * 64, 64), :])
    pltpu.sync_copy(den2_h.at[0], d0)
    pltpu.sync_copy(den2_h.at[1], d1)

    def dcomb(z, carry):
        d0[pl.ds(z * 16, 16)] = d0[pl.ds(z * 16, 16)] + d1[pl.ds(z * 16, 16)]
        return carry

    lax.fori_loop(0, NPAD // 16, dcomb, 0)

    @pl.when(c == 0)
    def _():
        pltpu.sync_copy(bias_h.at[pl.ds(0, Hh)], bias_v)

    @pl.when(c == 1)
    def _():
        pltpu.sync_copy(bias_h.at[pl.ds(Hh, Hh)], bias_v)

    plsc.subcore_barrier()

    def chunk(i, carry):
        eb = s * EPT + i * B
        pltpu.sync_copy(src_h.at[pl.ds(eb, B)], src_v)
        pltpu.sync_copy(dst_h.at[pl.ds(eb, B)], dst_v)
        pltpu.sync_copy(ex_h.at[pl.ds(eb, B)], ex_c)

        @pl.when(c == 0)
        def _():
            pltpu.async_copy(xllo_h.at[src_v], xj, sem).wait()

        @pl.when(c == 1)
        def _():
            pltpu.async_copy(xlhi_h.at[src_v], xj, sem).wait()

        def group(g, gcarry):
            idx16 = dst_v[pl.ds(g * 16, 16)]
            denv = plsc.load_gather(d0, [idx16])
            al = ex_c[pl.ds(g * 16, 16)] / (denv + 1e-16)
            for j in range(16):
                e = g * 16 + j
                aj = _dyn_gather(al, lane * 0 + j)
                for k in range(nkh):
                    xj[e, pl.ds(k * 16, 16)] = xj[e, pl.ds(k * 16, 16)] * aj
            return gcarry

        lax.fori_loop(0, B // 16, group, 0)
        pltpu.sync_copy(xj, acc_sh.at[dst_v], add=True)
        return carry

    lax.fori_loop(0, EPT // B, chunk, 0)
    plsc.subcore_barrier()
    bk = [bias_v[pl.ds(k * 16, 16)] for k in range(nkh)]
    for z in range(10):
        rows = s * 640 + z * 64
        pltpu.sync_copy(acc_sh.at[pl.ds(rows, 64), :], wb)

        def rrow(r, carry):
            for k in range(nkh):
                wb[r, pl.ds(k * 16, 16)] = jnp.maximum(
                    wb[r, pl.ds(k * 16, 16)] + bk[k], 0.0)
            return carry

        lax.fori_loop(0, 64, rrow, 0)

        @pl.when(c == 0)
        def _():
            pltpu.sync_copy(wb, hlo_h.at[pl.ds(rows, 64), :])

        @pl.when(c == 1)
        def _():
            pltpu.sync_copy(wb, hhi_h.at[pl.ds(rows, 64), :])


def _sc_pass2(src, dst, ex, den2, xl_lo, xl_hi, bias):
    Hh = xl_lo.shape[1]
    kern = functools.partial(
        pl.kernel,
        mesh=_mesh(),
        compiler_params=pltpu.CompilerParams(needs_layout_passes=False),
        out_type=[
            jax.ShapeDtypeStruct((NPAD, Hh), jnp.float32),
            jax.ShapeDtypeStruct((NPAD, Hh), jnp.float32),
        ],
        scratch_types=[
            pltpu.VMEM((B,), jnp.int32),
            pltpu.VMEM((B,), jnp.int32),
            pltpu.VMEM((B,), jnp.float32),
            pltpu.VMEM((B, Hh), jnp.float32),
            pltpu.VMEM((NPAD,), jnp.float32),
            pltpu.VMEM((NPAD,), jnp.float32),
            pltpu.VMEM((64, Hh), jnp.float32),
            pltpu.VMEM((Hh,), jnp.float32),
            pltpu.VMEM_SHARED((NPAD, Hh), jnp.float32),
            pltpu.SemaphoreType.DMA,
        ],
    )
    return kern(functools.partial(_sc_pass2_body, Hh))(
        src, dst, ex, den2, xl_lo, xl_hi, bias)


def _sc_pass2f_body(Hf, src_h, dst_h, ex_h, den2_h, xl_h, h0_h, h1_h,
                    src_v, dst_v, ex_c, xj, d0, d1, wb, acc_sh, sem):
    nk = Hf // 16
    c = lax.axis_index("c")
    s = lax.axis_index("s")
    lane = lax.iota(jnp.int32, 16)

    def zrow(r, carry):
        for k in range(nk):
            wb[r, pl.ds(k * 16, 16)] = jnp.zeros((16,), jnp.float32)
        return carry

    lax.fori_loop(0, 64, zrow, 0)
    for z in range(10):
        pltpu.sync_copy(wb, acc_sh.at[pl.ds(s * 640 + z * 64, 64), :])
    pltpu.sync_copy(den2_h.at[0], d0)
    pltpu.sync_copy(den2_h.at[1], d1)

    def dcomb(z, carry):
        d0[pl.ds(z * 16, 16)] = d0[pl.ds(z * 16, 16)] + d1[pl.ds(z * 16, 16)]
        return carry

    lax.fori_loop(0, NPAD // 16, dcomb, 0)
    plsc.subcore_barrier()
    base0 = (c * 16 + s) * EPW

    def chunk(i, carry):
        eb = base0 + i * B
        pltpu.sync_copy(src_h.at[pl.ds(eb, B)], src_v)
        pltpu.sync_copy(dst_h.at[pl.ds(eb, B)], dst_v)
        pltpu.sync_copy(ex_h.at[pl.ds(eb, B)], ex_c)
        pltpu.async_copy(xl_h.at[src_v], xj, sem).wait()

        def group(g, gcarry):
            idx16 = dst_v[pl.ds(g * 16, 16)]
            denv = plsc.load_gather(d0, [idx16])
            al = ex_c[pl.ds(g * 16, 16)] / (denv + 1e-16)
            for j in range(16):
                e = g * 16 + j
                aj = _dyn_gather(al, lane * 0 + j)
                for k in range(nk):
                    xj[e, pl.ds(k * 16, 16)] = xj[e, pl.ds(k * 16, 16)] * aj
            return gcarry

        lax.fori_loop(0, B // 16, group, 0)
        pltpu.sync_copy(xj, acc_sh.at[dst_v], add=True)
        return carry

    lax.fori_loop(0, EPW // B, chunk, 0)
    plsc.subcore_barrier()
    for z in range(10):
        rows = s * 640 + z * 64
        pltpu.sync_copy(acc_sh.at[pl.ds(rows, 64), :], wb)

        @pl.when(c == 0)
        def _():
            pltpu.sync_copy(wb, h0_h.at[pl.ds(rows, 64), :])

        @pl.when(c == 1)
        def _():
            pltpu.sync_copy(wb, h1_h.at[pl.ds(rows, 64), :])


def _sc_pass2_full(src, dst, ex, den2, xl):
    Hf = xl.shape[1]
    kern = functools.partial(
        pl.kernel,
        mesh=_mesh(),
        compiler_params=pltpu.CompilerParams(needs_layout_passes=False),
        out_type=[
            jax.ShapeDtypeStruct((NPAD, Hf), jnp.float32),
            jax.ShapeDtypeStruct((NPAD, Hf), jnp.float32),
        ],
        scratch_types=[
            pltpu.VMEM((B,), jnp.int32),
            pltpu.VMEM((B,), jnp.int32),
            pltpu.VMEM((B,), jnp.float32),
            pltpu.VMEM((B, Hf), jnp.float32),
            pltpu.VMEM((NPAD,), jnp.float32),
            pltpu.VMEM((NPAD,), jnp.float32),
            pltpu.VMEM((64, Hf), jnp.float32),
            pltpu.VMEM_SHARED((NPAD, Hf), jnp.float32),
            pltpu.SemaphoreType.DMA,
        ],
    )
    return kern(functools.partial(_sc_pass2f_body, Hf))(src, dst, ex, den2, xl)


def _head_body(hp0_ref, hp1_ref, bias2_ref, batch_ref, wd1_ref, bd1_ref,
               gamma_ref, beta_ref, wd2_ref, bd2_ref, out_ref):
    h = hp0_ref[...] + hp1_ref[...] + bias2_ref[...]
    h = jnp.maximum(h, 0.0)
    b = batch_ref[...]
    gids = lax.broadcasted_iota(jnp.int32, (G, NPAD), 0)
    onehot = (b == gids).astype(jnp.float32)
    sums = jnp.dot(onehot, h, preferred_element_type=jnp.float32,
                   precision=lax.Precision.HIGHEST)
    cnts = jnp.sum(onehot, axis=1, keepdims=True)
    p = sums / jnp.maximum(cnts, 1.0)
    p = jnp.dot(p, wd1_ref[...], preferred_element_type=jnp.float32) + bd1_ref[...]
    mean = jnp.mean(p, axis=0, keepdims=True)
    var = jnp.mean((p - mean) ** 2, axis=0, keepdims=True)
    p = (p - mean) / jnp.sqrt(var + 1e-5) * gamma_ref[...] + beta_ref[...]
    p = jnp.where(p >= 0, p, 0.1 * p)
    out_ref[...] = jnp.dot(p, wd2_ref[...], preferred_element_type=jnp.float32) + bd2_ref[...]


def _head(hp0, hp1, bias2, batch, Wd1, bd1, gamma, beta, Wd2, bd2):
    b_pad = jnp.full((1, NPAD), G, jnp.int32).at[0, :N].set(batch)
    return pl.pallas_call(
        _head_body,
        out_shape=jax.ShapeDtypeStruct((G, Wd2.shape[1]), jnp.float32),
    )(hp0, hp1, bias2.reshape(1, -1), b_pad, Wd1, bd1.reshape(1, -1),
      gamma.reshape(1, -1), beta.reshape(1, -1), Wd2, bd2.reshape(1, -1))


def kernel(x, edge_index, edge_attr, batch,
           Wl1, bl1, Wr1, br1, We1, att1, bias1,
           Wl2, bl2, Wr2, br2, We2, att2, bias2,
           Wd1, bd1, gamma, beta, Wd2, bd2):
    src = edge_index[0]
    dst = edge_index[1]
    x_pad = jnp.zeros((NPAD, x.shape[1]), x.dtype).at[:N].set(x)

    xl1, xr1, xl_lo, xl_hi = _node_mm1(x_pad, Wl1, bl1, Wr1, br1)
    eW1 = _edge_mm(edge_attr, We1)
    ex1, den1 = _sc_pass1(src, dst, xl1, xr1, eW1, att1)
    h1_lo, h1_hi = _sc_pass2(src, dst, ex1, den1, xl_lo, xl_hi, bias1)

    xl2, xr2 = _node_mm2(h1_lo, h1_hi, Wl2, bl2, Wr2, br2)
    eW2 = _edge_mm(edge_attr, We2)
    ex2, den2 = _sc_pass1(src, dst, xl2, xr2, eW2, att2)
    hp0, hp1 = _sc_pass2_full(src, dst, ex2, den2, xl2)

    return _head(hp0, hp1, bias2, batch, Wd1, bd1, gamma, beta, Wd2, bd2)


# speedup vs baseline: 5.5820x; 1.6088x over previous
import functools

import jax
import jax.numpy as jnp
from jax import lax
from jax.experimental import pallas as pl
from jax.experimental.pallas import tpu as pltpu
from jax.experimental.pallas import tpu_sc as plsc

N = 10000
E = 320000
G = 64
NPAD = 10240
NW = 32
B = 80
EPW = E // NW
EPT = E // 16

_mesh = lambda: plsc.VectorSubcoreMesh(core_axis_name="c", subcore_axis_name="s")


def _dyn_gather(v, idx):
    return lax.gather(
        v, idx[:, None],
        dimension_numbers=lax.GatherDimensionNumbers(
            offset_dims=(), collapsed_slice_dims=(0,), start_index_map=(0,)),
        slice_sizes=(1,),
        mode=lax.GatherScatterMode.PROMISE_IN_BOUNDS)


def _lane_sum(v, lane):
    for off in (8, 4, 2, 1):
        v = v + _dyn_gather(v, lax.bitwise_xor(lane, off))
    return v


def _node_mm1_body(x_ref, wl_ref, wr_ref, bl_ref, br_ref,
                   xllo, xlhi, xrlo, xrhi):
    xb = x_ref[...]
    xl = jnp.dot(xb, wl_ref[...], preferred_element_type=jnp.float32) + bl_ref[...]
    xr = jnp.dot(xb, wr_ref[...], preferred_element_type=jnp.float32) + br_ref[...]
    h = xl.shape[1] // 2
    xllo[...] = xl[:, :h]
    xlhi[...] = xl[:, h:]
    xrlo[...] = xr[:, :h]
    xrhi[...] = xr[:, h:]


def _node_mm1(x_pad, Wl, bl, Wr, br):
    D, H = Wl.shape
    Hh = H // 2
    R = 1024
    return pl.pallas_call(
        _node_mm1_body,
        grid=(NPAD // R,),
        in_specs=[
            pl.BlockSpec((R, D), lambda i: (i, 0)),
            pl.BlockSpec((D, H), lambda i: (0, 0)),
            pl.BlockSpec((D, H), lambda i: (0, 0)),
            pl.BlockSpec((1, H), lambda i: (0, 0)),
            pl.BlockSpec((1, H), lambda i: (0, 0)),
        ],
        out_specs=[pl.BlockSpec((R, Hh), lambda i: (i, 0))] * 4,
        out_shape=[jax.ShapeDtypeStruct((NPAD, Hh), jnp.float32)] * 4,
    )(x_pad, Wl, Wr, bl.reshape(1, -1), br.reshape(1, -1))


def _node_mm2_body(ha_ref, hb_ref, wla_ref, wlb_ref, wra_ref, wrb_ref,
                   bl_ref, br_ref, xlf, xrf):
    ha = ha_ref[...]
    hb = hb_ref[...]
    xlf[...] = (jnp.dot(ha, wla_ref[...], preferred_element_type=jnp.float32)
                + jnp.dot(hb, wlb_ref[...], preferred_element_type=jnp.float32)
                + bl_ref[...])
    xrf[...] = (jnp.dot(ha, wra_ref[...], preferred_element_type=jnp.float32)
                + jnp.dot(hb, wrb_ref[...], preferred_element_type=jnp.float32)
                + br_ref[...])


def _node_mm2(h_lo, h_hi, Wl, bl, Wr, br):
    Dh = h_lo.shape[1]
    H = Wl.shape[1]
    R = 1024
    return pl.pallas_call(
        _node_mm2_body,
        grid=(NPAD // R,),
        in_specs=[
            pl.BlockSpec((R, Dh), lambda i: (i, 0)),
            pl.BlockSpec((R, Dh), lambda i: (i, 0)),
            pl.BlockSpec((Dh, H), lambda i: (0, 0)),
            pl.BlockSpec((Dh, H), lambda i: (0, 0)),
            pl.BlockSpec((Dh, H), lambda i: (0, 0)),
            pl.BlockSpec((Dh, H), lambda i: (0, 0)),
            pl.BlockSpec((1, H), lambda i: (0, 0)),
            pl.BlockSpec((1, H), lambda i: (0, 0)),
        ],
        out_specs=[pl.BlockSpec((R, H), lambda i: (i, 0))] * 2,
        out_shape=[jax.ShapeDtypeStruct((NPAD, H), jnp.float32)] * 2,
    )(h_lo, h_hi, Wl[:Dh], Wl[Dh:], Wr[:Dh], Wr[Dh:],
      bl.reshape(1, -1), br.reshape(1, -1))


def _edge_mm_body(ea_ref, we_ref, out_ref):
    out_ref[...] = jnp.dot(ea_ref[...], we_ref[...],
                           preferred_element_type=jnp.float32)


def _edge_mm(edge_attr, We):
    De, H = We.shape
    R = 3200
    return pl.pallas_call(
        _edge_mm_body,
        grid=(E // R,),
        in_specs=[
            pl.BlockSpec((R, De), lambda i: (i, 0)),
            pl.BlockSpec((De, H), lambda i: (0, 0)),
        ],
        out_specs=pl.BlockSpec((R, H), lambda i: (i, 0)),
        out_shape=jax.ShapeDtypeStruct((E, H), jnp.float32),
    )(edge_attr, We)


def _edge_mm_half_body(ea_ref, we_ref, olo_ref, ohi_ref):
    r = jnp.dot(ea_ref[...], we_ref[...], preferred_element_type=jnp.float32)
    h = r.shape[1] // 2
    olo_ref[...] = r[:, :h]
    ohi_ref[...] = r[:, h:]


def _edge_mm_half(edge_attr, We):
    De, H = We.shape
    Hh = H // 2
    R = 3200
    return pl.pallas_call(
        _edge_mm_half_body,
        grid=(E // R,),
        in_specs=[
            pl.BlockSpec((R, De), lambda i: (i, 0)),
            pl.BlockSpec((De, H), lambda i: (0, 0)),
        ],
        out_specs=[pl.BlockSpec((R, Hh), lambda i: (i, 0))] * 2,
        out_shape=[jax.ShapeDtypeStruct((E, Hh), jnp.float32)] * 2,
    )(edge_attr, We)


def _sc_pass1s_body(src_h, dst_h, xllo_h, xlhi_h, xrlo_h, xrhi_h,
                    ewlo_h, ewhi_h, att_h, ex_h, den_h,
                    sv0, dv0, sv1, dv1, xj0, xi0, ew0, xj1, xi1, ew1,
                    att_v, ex_mb, lbuf, macc, den_sh, sem0, sem1):
    c = lax.axis_index("c")
    s = lax.axis_index("s")
    wid = c * 16 + s
    base = wid * EPW
    lane = lax.iota(jnp.int32, 16)
    for z in range(40):
        macc[pl.ds(z * 16, 16)] = jnp.zeros((16,), jnp.float32)
    pltpu.sync_copy(macc, den_sh.at[pl.ds(s * 640, 640)])
    pltpu.sync_copy(att_h, att_v)
    plsc.subcore_barrier()
    attk = [att_v[pl.ds(k * 16, 16)] for k in range(16)]

    def issue(i, lo, sv, dv, xj, xi, ew, sm):
        eb = base + i * B
        pltpu.sync_copy(src_h.at[pl.ds(eb, B)], sv)
        pltpu.sync_copy(dst_h.at[pl.ds(eb, B)], dv)
        pltpu.async_copy((xllo_h if lo else xlhi_h).at[sv], xj, sm)
        pltpu.async_copy((xrlo_h if lo else xrhi_h).at[dv], xi, sm)
        pltpu.async_copy((ewlo_h if lo else ewhi_h).at[pl.ds(eb, B), :], ew, sm)

    def drain(lo, sv, dv, xj, xi, ew, sm):
        pltpu.make_async_copy((xllo_h if lo else xlhi_h).at[sv], xj, sm).wait()
        pltpu.make_async_copy((xrlo_h if lo else xrhi_h).at[dv], xi, sm).wait()
        pltpu.make_async_copy((ewlo_h if lo else ewhi_h).at[pl.ds(0, B), :],
                              ew, sm).wait()

    def half_logits(xj, xi, ew, koff, g, init):
        lvec = init
        for j in range(16):
            e = g * 16 + j
            acc = jnp.zeros((16,), jnp.float32)
            for k in range(8):
                v = (xj[e, pl.ds(k * 16, 16)]
                     + xi[e, pl.ds(k * 16, 16)]
                     + ew[e, pl.ds(k * 16, 16)])
                m = jnp.maximum(v, 0.2 * v)
                acc = acc + m * attk[koff + k]
            lvec = jnp.where(lane == j, lvec + _lane_sum(acc, lane), lvec)
        return lvec

    def chunk(i2, carry):
        drain(True, sv0, dv0, xj0, xi0, ew0, sem0)
        issue(i2, False, sv1, dv1, xj1, xi1, ew1, sem1)

        def grp_lo(g, gc):
            lbuf[pl.ds(g * 16, 16)] = half_logits(
                xj0, xi0, ew0, 0, g, jnp.zeros((16,), jnp.float32))
            return gc

        lax.fori_loop(0, B // 16, grp_lo, 0)
        drain(False, sv1, dv1, xj1, xi1, ew1, sem1)
        nxt = jnp.minimum(i2 + 1, EPW // B - 1)
        issue(nxt, True, sv0, dv0, xj0, xi0, ew0, sem0)

        def grp_hi(g, gc):
            lvec = half_logits(xj1, xi1, ew1, 8, g, lbuf[pl.ds(g * 16, 16)])
            ex_mb[pl.ds(g * 16, 16)] = jnp.exp(lvec)
            return gc

        lax.fori_loop(0, B // 16, grp_hi, 0)
        pltpu.sync_copy(ex_mb, den_sh.at[dv1], add=True)
        pltpu.sync_copy(ex_mb, ex_h.at[pl.ds(base + i2 * B, B)])
        return carry

    issue(0, True, sv0, dv0, xj0, xi0, ew0, sem0)
    lax.fori_loop(0, EPW // B, chunk, 0)
    drain(True, sv0, dv0, xj0, xi0, ew0, sem0)
    plsc.subcore_barrier()
    pltpu.sync_copy(den_sh.at[pl.ds(s * 640, 640)], macc)
    pltpu.sync_copy(macc, den_h.at[c, pl.ds(s * 640, 640)])


def _sc_pass1_split(src, dst, xl_lo, xl_hi, xr_lo, xr_hi, ew_lo, ew_hi, att):
    Hh = ew_lo.shape[1]
    kern = functools.partial(
        pl.kernel,
        mesh=_mesh(),
        compiler_params=pltpu.CompilerParams(needs_layout_passes=False),
        out_type=[
            jax.ShapeDtypeStruct((E,), jnp.float32),
            jax.ShapeDtypeStruct((2, NPAD), jnp.float32),
        ],
        scratch_types=[
            pltpu.VMEM((B,), jnp.int32),
            pltpu.VMEM((B,), jnp.int32),
            pltpu.VMEM((B,), jnp.int32),
            pltpu.VMEM((B,), jnp.int32),
            pltpu.VMEM((B, Hh), jnp.float32),
            pltpu.VMEM((B, Hh), jnp.float32),
            pltpu.VMEM((B, Hh), jnp.float32),
            pltpu.VMEM((B, Hh), jnp.float32),
            pltpu.VMEM((B, Hh), jnp.float32),
            pltpu.VMEM((B, Hh), jnp.float32),
            pltpu.VMEM((2 * Hh,), jnp.float32),
            pltpu.VMEM((B,), jnp.float32),
            pltpu.VMEM((B,), jnp.float32),
            pltpu.VMEM((640,), jnp.float32),
            pltpu.VMEM_SHARED((NPAD,), jnp.float32),
            pltpu.SemaphoreType.DMA,
            pltpu.SemaphoreType.DMA,
        ],
    )
    return kern(_sc_pass1s_body)(src, dst, xl_lo, xl_hi, xr_lo, xr_hi,
                                 ew_lo, ew_hi, att)

def _sc_pass1_body(H, packed, src_h, dst_h, xl_h, xr_h, ew_h,
                   attc_h, ex_h, den_h,
                   sv0, dv0, sv1, dv1, xj0, xi0, ew0, xj1, xi1, ew1,
                   att_v, ex_mb, macc, den_sh, sem0, sem1):
    c = lax.axis_index("c")
    s = lax.axis_index("s")
    wid = c * 16 + s
    base = wid * EPW
    lane = lax.iota(jnp.int32, 16)
    for z in range(40):
        macc[pl.ds(z * 16, 16)] = jnp.zeros((16,), jnp.float32)
    pltpu.sync_copy(macc, den_sh.at[pl.ds(s * 640, 640)])
    pltpu.sync_copy(attc_h, att_v)
    plsc.subcore_barrier()
    if packed:
        nb = H // 32
        attev = [att_v[pl.ds(k * 16, 16)] for k in range(nb)]
        attod = [att_v[pl.ds(H // 2 + k * 16, 16)] for k in range(nb)]
    else:
        nb = H // 16
        attk = [att_v[pl.ds(k * 16, 16)] for k in range(nb)]

    def issue(i, sv, dv, xj, xi, ew, sm):
        eb = base + i * B
        pltpu.sync_copy(src_h.at[pl.ds(eb, B)], sv)
        pltpu.sync_copy(dst_h.at[pl.ds(eb, B)], dv)
        pltpu.async_copy(xl_h.at[sv], xj, sm)
        pltpu.async_copy(xr_h.at[dv], xi, sm)
        pltpu.async_copy(ew_h.at[pl.ds(eb, B), :], ew, sm)

    def drain(sv, dv, xj, xi, ew, sm):
        pltpu.make_async_copy(xl_h.at[sv], xj, sm).wait()
        pltpu.make_async_copy(xr_h.at[dv], xi, sm).wait()
        pltpu.make_async_copy(ew_h.at[pl.ds(0, B), :], ew, sm).wait()

    def compute(xj, xi, ew, dv, half):
        def group(g, gcarry):
            lvec = jnp.zeros((16,), jnp.float32)
            for j in range(16):
                e = g * 16 + j
                acc = jnp.zeros((16,), jnp.float32)
                for k in range(nb):
                    if packed:
                        xje, xjo = plsc.unpack(
                            plsc.bitcast(xj[e, pl.ds(k * 16, 16)], jnp.bfloat16),
                            format=plsc.PackFormat.INTERLEAVED)
                        xie, xio = plsc.unpack(
                            plsc.bitcast(xi[e, pl.ds(k * 16, 16)], jnp.bfloat16),
                            format=plsc.PackFormat.INTERLEAVED)
                        ewe, ewo = plsc.unpack(
                            plsc.bitcast(ew[e, pl.ds(k * 16, 16)], jnp.bfloat16),
                            format=plsc.PackFormat.INTERLEAVED)
                        ve = xje + xie + ewe
                        vo = xjo + xio + ewo
                        me = jnp.maximum(ve, 0.2 * ve)
                        mo = jnp.maximum(vo, 0.2 * vo)
                        acc = acc + me * attev[k] + mo * attod[k]
                    else:
                        v = (xj[e, pl.ds(k * 16, 16)]
                             + xi[e, pl.ds(k * 16, 16)]
                             + ew[e, pl.ds(k * 16, 16)])
                        m = jnp.maximum(v, 0.2 * v)
                        acc = acc + m * attk[k]
                lvec = jnp.where(lane == j, _lane_sum(acc, lane), lvec)
            exv = jnp.exp(lvec)
            ex_mb[pl.ds(half * B + g * 16, 16)] = exv
            return gcarry

        lax.fori_loop(0, B // 16, group, 0)
        pltpu.sync_copy(ex_mb.at[pl.ds(half * B, B)],
                        den_sh.at[dv], add=True)

    issue(0, sv0, dv0, xj0, xi0, ew0, sem0)

    def pair(i2, carry):
        i = 2 * i2
        drain(sv0, dv0, xj0, xi0, ew0, sem0)
        issue(i + 1, sv1, dv1, xj1, xi1, ew1, sem1)
        compute(xj0, xi0, ew0, dv0, 0)
        drain(sv1, dv1, xj1, xi1, ew1, sem1)
        issue(i + 2, sv0, dv0, xj0, xi0, ew0, sem0)
        compute(xj1, xi1, ew1, dv1, 1)
        pltpu.sync_copy(ex_mb, ex_h.at[pl.ds(base + i * B, 2 * B)])
        return carry

    npair = (EPW // B) // 2
    lax.fori_loop(0, npair, pair, 0)
    drain(sv0, dv0, xj0, xi0, ew0, sem0)
    compute(xj0, xi0, ew0, dv0, 0)
    pltpu.sync_copy(ex_mb.at[pl.ds(0, B)],
                    ex_h.at[pl.ds(base + (EPW // B - 1) * B, B)])

    plsc.subcore_barrier()
    pltpu.sync_copy(den_sh.at[pl.ds(s * 640, 640)], macc)
    pltpu.sync_copy(macc, den_h.at[c, pl.ds(s * 640, 640)])


def _sc_pass1(src, dst, xl, xr, eW, attc, packed):
    W = eW.shape[1]
    H = W * 2 if packed else W
    kern = functools.partial(
        pl.kernel,
        mesh=_mesh(),
        compiler_params=pltpu.CompilerParams(needs_layout_passes=False),
        out_type=[
            jax.ShapeDtypeStruct((E,), jnp.float32),
            jax.ShapeDtypeStruct((2, NPAD), jnp.float32),
        ],
        scratch_types=[
            pltpu.VMEM((B,), jnp.int32),
            pltpu.VMEM((B,), jnp.int32),
            pltpu.VMEM((B,), jnp.int32),
            pltpu.VMEM((B,), jnp.int32),
            pltpu.VMEM((B, W), jnp.float32),
            pltpu.VMEM((B, W), jnp.float32),
            pltpu.VMEM((B, W), jnp.float32),
            pltpu.VMEM((B, W), jnp.float32),
            pltpu.VMEM((B, W), jnp.float32),
            pltpu.VMEM((B, W), jnp.float32),
            pltpu.VMEM((H,), jnp.float32),
            pltpu.VMEM((2 * B,), jnp.float32),
            pltpu.VMEM((640,), jnp.float32),
            pltpu.VMEM_SHARED((NPAD,), jnp.float32),
            pltpu.SemaphoreType.DMA,
            pltpu.SemaphoreType.DMA,
        ],
    )
    return kern(functools.partial(_sc_pass1_body, H, packed))(
        src, dst, xl, xr, eW, attc)


def _sc_pass2_body(Hh, src_h, dst_h, ex_h, den2_h, xllo_h, xlhi_h, bias_h,
                   hlo_h, hhi_h,
                   src_v, dst_v, ex_c, xj, d0, d1, wb, bias_v, acc_sh, sem):
    nkh = Hh // 16
    c = lax.axis_index("c")
    s = lax.axis_index("s")
    lane = lax.iota(jnp.int32, 16)

    def zrow(r, carry):
        for k in range(nkh):
            wb[r, pl.ds(k * 16, 16)] = jnp.zeros((16,), jnp.float32)
        return carry

    lax.fori_loop(0, 64, zrow, 0)
    for z in range(10):
        pltpu.sync_copy(wb, acc_sh.at[pl.ds(s * 640 + z * 64, 64), :])
    pltpu.sync_copy(den2_h.at[0], d0)
    pltpu.sync_copy(den2_h.at[1], d1)

    def dcomb(z, carry):
        d0[pl.ds(z * 16, 16)] = d0[pl.ds(z * 16, 16)] + d1[pl.ds(z * 16, 16)]
        return carry

    lax.fori_loop(0, NPAD // 16, dcomb, 0)

    @pl.when(c == 0)
    def _():
        pltpu.sync_copy(bias_h.at[pl.ds(0, Hh)], bias_v)

    @pl.when(c == 1)
    def _():
        pltpu.sync_copy(bias_h.at[pl.ds(Hh, Hh)], bias_v)

    plsc.subcore_barrier()

    def chunk(i, carry):
        eb = s * EPT + i * B
        pltpu.sync_copy(src_h.at[pl.ds(eb, B)], src_v)
        pltpu.sync_copy(dst_h.at[pl.ds(eb, B)], dst_v)
        pltpu.sync_copy(ex_h.at[pl.ds(eb, B)], ex_c)

        @pl.when(c == 0)
        def _():
            pltpu.async_copy(xllo_h.at[src_v], xj, sem).wait()

        @pl.when(c == 1)
        def _():
            pltpu.async_copy(xlhi_h.at[src_v], xj, sem).wait()

        def group(g, gcarry):
            idx16 = dst_v[pl.ds(g * 16, 16)]
            denv = plsc.load_gather(d0, [idx16])
            al = ex_c[pl.ds(g * 16, 16)] / (denv + 1e-16)
            for j in range(16):
                e = g * 16 + j
                aj = _dyn_gather(al, lane * 0 + j)
                for k in range(nkh):
                    xj[e, pl.ds(k * 16, 16)] = xj[e, pl.ds(k * 16, 16)] * aj
            return gcarry

        lax.fori_loop(0, B // 16, group, 0)
        pltpu.sync_copy(xj, acc_sh.at[dst_v], add=True)
        return carry

    lax.fori_loop(0, EPT // B, chunk, 0)
    plsc.subcore_barrier()
    bk = [bias_v[pl.ds(k * 16, 16)] for k in range(nkh)]
    for z in range(10):
        rows = s * 640 + z * 64
        pltpu.sync_copy(acc_sh.at[pl.ds(rows, 64), :], wb)

        def rrow(r, carry):
            for k in range(nkh):
                wb[r, pl.ds(k * 16, 16)] = jnp.maximum(
                    wb[r, pl.ds(k * 16, 16)] + bk[k], 0.0)
            return carry

        lax.fori_loop(0, 64, rrow, 0)

        @pl.when(c == 0)
        def _():
            pltpu.sync_copy(wb, hlo_h.at[pl.ds(rows, 64), :])

        @pl.when(c == 1)
        def _():
            pltpu.sync_copy(wb, hhi_h.at[pl.ds(rows, 64), :])


def _sc_pass2(src, dst, ex, den2, xl_lo, xl_hi, bias):
    Hh = xl_lo.shape[1]
    kern = functools.partial(
        pl.kernel,
        mesh=_mesh(),
        compiler_params=pltpu.CompilerParams(needs_layout_passes=False),
        out_type=[
            jax.ShapeDtypeStruct((NPAD, Hh), jnp.float32),
            jax.ShapeDtypeStruct((NPAD, Hh), jnp.float32),
        ],
        scratch_types=[
            pltpu.VMEM((B,), jnp.int32),
            pltpu.VMEM((B,), jnp.int32),
            pltpu.VMEM((B,), jnp.float32),
            pltpu.VMEM((B, Hh), jnp.float32),
            pltpu.VMEM((NPAD,), jnp.float32),
            pltpu.VMEM((NPAD,), jnp.float32),
            pltpu.VMEM((64, Hh), jnp.float32),
            pltpu.VMEM((Hh,), jnp.float32),
            pltpu.VMEM_SHARED((NPAD, Hh), jnp.float32),
            pltpu.SemaphoreType.DMA,
        ],
    )
    return kern(functools.partial(_sc_pass2_body, Hh))(
        src, dst, ex, den2, xl_lo, xl_hi, bias)


def _sc_pass2f_body(Hf, src_h, dst_h, ex_h, den2_h, xl_h, h0_h, h1_h,
                    src_v, dst_v, ex_c, xj, d0, d1, wb, acc_sh, sem):
    nk = Hf // 16
    c = lax.axis_index("c")
    s = lax.axis_index("s")
    lane = lax.iota(jnp.int32, 16)

    def zrow(r, carry):
        for k in range(nk):
            wb[r, pl.ds(k * 16, 16)] = jnp.zeros((16,), jnp.float32)
        return carry

    lax.fori_loop(0, 64, zrow, 0)
    for z in range(10):
        pltpu.sync_copy(wb, acc_sh.at[pl.ds(s * 640 + z * 64, 64), :])
    pltpu.sync_copy(den2_h.at[0], d0)
    pltpu.sync_copy(den2_h.at[1], d1)

    def dcomb(z, carry):
        d0[pl.ds(z * 16, 16)] = d0[pl.ds(z * 16, 16)] + d1[pl.ds(z * 16, 16)]
        return carry

    lax.fori_loop(0, NPAD // 16, dcomb, 0)
    plsc.subcore_barrier()
    base0 = (c * 16 + s) * EPW

    def chunk(i, carry):
        eb = base0 + i * B
        pltpu.sync_copy(src_h.at[pl.ds(eb, B)], src_v)
        pltpu.sync_copy(dst_h.at[pl.ds(eb, B)], dst_v)
        pltpu.sync_copy(ex_h.at[pl.ds(eb, B)], ex_c)
        pltpu.async_copy(xl_h.at[src_v], xj, sem).wait()

        def group(g, gcarry):
            idx16 = dst_v[pl.ds(g * 16, 16)]
            denv = plsc.load_gather(d0, [idx16])
            al = ex_c[pl.ds(g * 16, 16)] / (denv + 1e-16)
            for j in range(16):
                e = g * 16 + j
                aj = _dyn_gather(al, lane * 0 + j)
                for k in range(nk):
                    xj[e, pl.ds(k * 16, 16)] = xj[e, pl.ds(k * 16, 16)] * aj
            return gcarry

        lax.fori_loop(0, B // 16, group, 0)
        pltpu.sync_copy(xj, acc_sh.at[dst_v], add=True)
        return carry

    lax.fori_loop(0, EPW // B, chunk, 0)
    plsc.subcore_barrier()
    for z in range(10):
        rows = s * 640 + z * 64
        pltpu.sync_copy(acc_sh.at[pl.ds(rows, 64), :], wb)

        @pl.when(c == 0)
        def _():
            pltpu.sync_copy(wb, h0_h.at[pl.ds(rows, 64), :])

        @pl.when(c == 1)
        def _():
            pltpu.sync_copy(wb, h1_h.at[pl.ds(rows, 64), :])


def _sc_pass2_full(src, dst, ex, den2, xl):
    Hf = xl.shape[1]
    kern = functools.partial(
        pl.kernel,
        mesh=_mesh(),
        compiler_params=pltpu.CompilerParams(needs_layout_passes=False),
        out_type=[
            jax.ShapeDtypeStruct((NPAD, Hf), jnp.float32),
            jax.ShapeDtypeStruct((NPAD, Hf), jnp.float32),
        ],
        scratch_types=[
            pltpu.VMEM((B,), jnp.int32),
            pltpu.VMEM((B,), jnp.int32),
            pltpu.VMEM((B,), jnp.float32),
            pltpu.VMEM((B, Hf), jnp.float32),
            pltpu.VMEM((NPAD,), jnp.float32),
            pltpu.VMEM((NPAD,), jnp.float32),
            pltpu.VMEM((64, Hf), jnp.float32),
            pltpu.VMEM_SHARED((NPAD, Hf), jnp.float32),
            pltpu.SemaphoreType.DMA,
        ],
    )
    return kern(functools.partial(_sc_pass2f_body, Hf))(src, dst, ex, den2, xl)


def _head_body(hp0_ref, hp1_ref, bias2_ref, batch_ref, wd1_ref, bd1_ref,
               gamma_ref, beta_ref, wd2_ref, bd2_ref, out_ref):
    h = hp0_ref[...] + hp1_ref[...] + bias2_ref[...]
    h = jnp.maximum(h, 0.0)
    b = batch_ref[...]
    gids = lax.broadcasted_iota(jnp.int32, (G, NPAD), 0)
    onehot = (b == gids).astype(jnp.float32)
    sums = jnp.dot(onehot, h, preferred_element_type=jnp.float32,
                   precision=lax.Precision.HIGHEST)
    cnts = jnp.sum(onehot, axis=1, keepdims=True)
    p = sums / jnp.maximum(cnts, 1.0)
    p = jnp.dot(p, wd1_ref[...], preferred_element_type=jnp.float32) + bd1_ref[...]
    mean = jnp.mean(p, axis=0, keepdims=True)
    var = jnp.mean((p - mean) ** 2, axis=0, keepdims=True)
    p = (p - mean) / jnp.sqrt(var + 1e-5) * gamma_ref[...] + beta_ref[...]
    p = jnp.where(p >= 0, p, 0.1 * p)
    out_ref[...] = jnp.dot(p, wd2_ref[...], preferred_element_type=jnp.float32) + bd2_ref[...]


def _head(hp0, hp1, bias2, batch, Wd1, bd1, gamma, beta, Wd2, bd2):
    b_pad = jnp.full((1, NPAD), G, jnp.int32).at[0, :N].set(batch)
    return pl.pallas_call(
        _head_body,
        out_shape=jax.ShapeDtypeStruct((G, Wd2.shape[1]), jnp.float32),
    )(hp0, hp1, bias2.reshape(1, -1), b_pad, Wd1, bd1.reshape(1, -1),
      gamma.reshape(1, -1), beta.reshape(1, -1), Wd2, bd2.reshape(1, -1))


def kernel(x, edge_index, edge_attr, batch,
           Wl1, bl1, Wr1, br1, We1, att1, bias1,
           Wl2, bl2, Wr2, br2, We2, att2, bias2,
           Wd1, bd1, gamma, beta, Wd2, bd2):
    src = edge_index[0]
    dst = edge_index[1]
    x_pad = jnp.zeros((NPAD, x.shape[1]), x.dtype).at[:N].set(x)

    xl_lo, xl_hi, xr_lo, xr_hi = _node_mm1(x_pad, Wl1, bl1, Wr1, br1)
    eW1lo, eW1hi = _edge_mm_half(edge_attr, We1)
    ex1, den1 = _sc_pass1_split(src, dst, xl_lo, xl_hi, xr_lo, xr_hi,
                                eW1lo, eW1hi, att1)
    h1_lo, h1_hi = _sc_pass2(src, dst, ex1, den1, xl_lo, xl_hi, bias1)

    xl2, xr2 = _node_mm2(h1_lo, h1_hi, Wl2, bl2, Wr2, br2)
    eW2 = _edge_mm(edge_attr, We2)
    ex2, den2 = _sc_pass1(src, dst, xl2, xr2, eW2, att2, packed=False)
    hp0, hp1 = _sc_pass2_full(src, dst, ex2, den2, xl2)

    return _head(hp0, hp1, bias2, batch, Wd1, bd1, gamma, beta, Wd2, bd2)
